# Initial kernel scaffold; baseline (speedup 1.0000x reference)
import jax
import jax.numpy as jnp
from jax.experimental import pallas as pl

N_TOKENS = 8192
HIDDEN = 32
BATCH = 4096
BM = 256


def _body(h_ref, c_ref, o_ref):
    c = c_ref[...]
    cn = jnp.sum(c * c, axis=1)
    dots = jax.lax.dot_general(
        h_ref[...], c,
        dimension_numbers=(((1,), (1,)), ((), ())),
        preferred_element_type=jnp.float32,
        precision=jax.lax.Precision.HIGHEST,
    )
    scores = cn[None, :] - 2.0 * dots
    minval = jnp.min(scores, axis=1, keepdims=True)
    iota = jax.lax.broadcasted_iota(jnp.int32, scores.shape, 1)
    idx = jnp.min(jnp.where(scores == minval, iota, N_TOKENS), axis=1)
    o_ref[...] = (iota == idx[:, None]).astype(jnp.float32)


def kernel(h, temperature, codebook):
    del temperature
    return pl.pallas_call(
        _body,
        grid=(BATCH // BM,),
        in_specs=[
            pl.BlockSpec((BM, HIDDEN), lambda i: (i, 0)),
            pl.BlockSpec((N_TOKENS, HIDDEN), lambda i: (0, 0)),
        ],
        out_specs=pl.BlockSpec((BM, N_TOKENS), lambda i: (i, 0)),
        out_shape=jax.ShapeDtypeStruct((BATCH, N_TOKENS), jnp.float32),
    )(h, codebook)


# speedup vs baseline: 3.9440x; 3.9440x over previous
import jax
import jax.numpy as jnp
from jax.experimental import pallas as pl

N_TOKENS = 8192
HIDDEN = 32
BATCH = 4096
BM = 256
KC = 512
NK = N_TOKENS // KC


def _body(h_ref, ct_ref, o_ref):
    h = h_ref[...]

    def step(k, carry):
        run_min, run_idx = carry
        ct = ct_ref[:, pl.ds(k * KC, KC)]
        cn = jnp.sum(ct * ct, axis=0)
        dots = jax.lax.dot_general(
            h, ct,
            dimension_numbers=(((1,), (0,)), ((), ())),
            preferred_element_type=jnp.float32,
            precision=jax.lax.Precision.HIGHEST,
        )
        scores = cn[None, :] - 2.0 * dots
        cmin = jnp.min(scores, axis=1, keepdims=True)
        iota = jax.lax.broadcasted_iota(jnp.int32, scores.shape, 1) + k * KC
        cidx = jnp.min(jnp.where(scores == cmin, iota, N_TOKENS),
                       axis=1, keepdims=True)
        better = cmin < run_min
        return (jnp.where(better, cmin, run_min),
                jnp.where(better, cidx, run_idx))

    run_min, run_idx = jax.lax.fori_loop(
        0, NK, step,
        (jnp.full((BM, 1), jnp.inf, jnp.float32),
         jnp.zeros((BM, 1), jnp.int32)))
    for k in range(NK):
        iota = jax.lax.broadcasted_iota(jnp.int32, (BM, KC), 1) + k * KC
        o_ref[:, k * KC:(k + 1) * KC] = (iota == run_idx).astype(jnp.float32)


def kernel(h, temperature, codebook):
    del temperature
    ct = codebook.T
    return pl.pallas_call(
        _body,
        grid=(BATCH // BM,),
        in_specs=[
            pl.BlockSpec((BM, HIDDEN), lambda i: (i, 0)),
            pl.BlockSpec((HIDDEN, N_TOKENS), lambda i: (0, 0)),
        ],
        out_specs=pl.BlockSpec((BM, N_TOKENS), lambda i: (i, 0)),
        out_shape=jax.ShapeDtypeStruct((BATCH, N_TOKENS), jnp.float32),
    )(h, ct)


# speedup vs baseline: 6.8332x; 1.7325x over previous
import jax
import jax.numpy as jnp
from jax.experimental import pallas as pl
from jax.experimental.pallas import tpu as pltpu

N_TOKENS = 8192
HIDDEN = 32
BATCH = 4096
BM = 256
KC = 512
NK = N_TOKENS // KC


def _body(h_ref, ct_ref, o_ref, s_ref):
    h1 = jnp.concatenate(
        [h_ref[...], jnp.ones((BM, 1), jnp.float32)], axis=1)

    cmin_cols = []
    for k in range(NK):
        ct = ct_ref[:, k * KC:(k + 1) * KC]
        cn = jnp.sum(ct * ct, axis=0)
        a = jnp.concatenate([ct * -2.0, cn[None, :]], axis=0)
        scores = jax.lax.dot_general(
            h1, a,
            dimension_numbers=(((1,), (0,)), ((), ())),
            preferred_element_type=jnp.float32,
            precision=jax.lax.Precision.HIGHEST,
        )
        s_ref[:, k * KC:(k + 1) * KC] = scores
        cmin_cols.append(jnp.min(scores, axis=1, keepdims=True))

    cmins = jnp.concatenate(cmin_cols, axis=1)
    gmin = jnp.min(cmins, axis=1, keepdims=True)
    iota = jax.lax.broadcasted_iota(jnp.int32, cmins.shape, 1)
    kwin = jnp.min(jnp.where(cmins == gmin, iota, NK),
                   axis=1, keepdims=True)

    for k in range(NK):
        scores = s_ref[:, k * KC:(k + 1) * KC]
        tgt = jnp.where(kwin == k, gmin, jnp.inf)
        o_ref[:, k * KC:(k + 1) * KC] = (scores == tgt).astype(jnp.float32)


def kernel(h, temperature, codebook):
    del temperature
    ct = codebook.T
    return pl.pallas_call(
        _body,
        grid=(BATCH // BM,),
        in_specs=[
            pl.BlockSpec((BM, HIDDEN), lambda i: (i, 0)),
            pl.BlockSpec((HIDDEN, N_TOKENS), lambda i: (0, 0)),
        ],
        out_specs=pl.BlockSpec((BM, N_TOKENS), lambda i: (i, 0)),
        out_shape=jax.ShapeDtypeStruct((BATCH, N_TOKENS), jnp.float32),
        scratch_shapes=[pltpu.VMEM((BM, N_TOKENS), jnp.float32)],
    )(h, ct)
